# Initial kernel scaffold; baseline (speedup 1.0000x reference)
import jax
import jax.numpy as jnp
from jax.experimental import pallas as pl


def kernel(edge_attr, W0, W1, W2):
    raise NotImplementedError("write your pallas kernel here")


# speedup vs baseline: 1.2634x; 1.2634x over previous
import functools

import jax
import jax.numpy as jnp
from jax import lax
from jax.experimental import pallas as pl
from jax.experimental.pallas import tpu as pltpu
from jax.experimental.pallas import tpu_sc as plsc

EMB = 256
CHUNK = 128
NUM_TILES = 32


def _prep_body(w0_ref, w1_ref, w2_ref, a0_ref, a1_ref, a2_ref, t_ref, c_ref):
    for a0 in range(w0_ref.shape[0]):
        for a1 in range(w1_ref.shape[0]):
            for a2 in range(w2_ref.shape[0]):
                c = (a0 * w1_ref.shape[0] + a1) * w2_ref.shape[0] + a2
                t_ref[c, :] = w0_ref[a0, :] + w1_ref[a1, :] + w2_ref[a2, :]
    n1 = w1_ref.shape[0]
    n2 = w2_ref.shape[0]
    c_ref[...] = (a0_ref[...] * n1 + a1_ref[...]) * n2 + a2_ref[...]


def _make_sc_gather(num_edges):
    nchunks = num_edges // CHUNK
    iters = (nchunks + NUM_TILES - 1) // NUM_TILES
    mesh = plsc.VectorSubcoreMesh(core_axis_name="c", subcore_axis_name="s")

    @functools.partial(
        pl.kernel,
        mesh=mesh,
        out_type=jax.ShapeDtypeStruct((num_edges, EMB), jnp.float32),
        scratch_types=[
            pltpu.VMEM((CHUNK,), jnp.int32),
            pltpu.VMEM((CHUNK, EMB), jnp.float32),
            pltpu.SemaphoreType.DMA,
        ],
    )
    def sc_gather(t_hbm, c_hbm, out_hbm, idx_v, rows_v, sem):
        wid = lax.axis_index("s") * 2 + lax.axis_index("c")

        def body(i, carry):
            g = wid + i * NUM_TILES

            @pl.when(g < nchunks)
            def _():
                base = g * CHUNK
                pltpu.sync_copy(c_hbm.at[pl.ds(base, CHUNK)], idx_v)
                pltpu.async_copy(t_hbm.at[idx_v], rows_v, sem).wait()
                pltpu.sync_copy(rows_v, out_hbm.at[pl.ds(base, CHUNK), :])

            return carry

        lax.fori_loop(0, iters, body, 0)

    return sc_gather


def kernel(edge_attr, W0, W1, W2):
    num_edges = edge_attr.shape[0]
    attr = edge_attr.astype(jnp.int32)
    rows = num_edges // CHUNK
    a0 = attr[:, 0].reshape(rows, CHUNK)
    a1 = attr[:, 1].reshape(rows, CHUNK)
    a2 = attr[:, 2].reshape(rows, CHUNK)

    ncombo = W0.shape[0] * W1.shape[0] * W2.shape[0]
    t, c2d = pl.pallas_call(
        _prep_body,
        out_shape=(
            jax.ShapeDtypeStruct((ncombo, EMB), jnp.float32),
            jax.ShapeDtypeStruct((rows, CHUNK), jnp.int32),
        ),
    )(W0, W1, W2, a0, a1, a2)
    c = c2d.reshape(num_edges)

    return _make_sc_gather(num_edges)(t, c)
